# baseline (speedup 1.0000x reference)
import functools

import jax
import jax.numpy as jnp
import numpy as np
from jax import lax
from jax.experimental import pallas as pl
from jax.experimental.pallas import tpu as pltpu
from jax.experimental.pallas import tpu_sc as plsc

_P1 = np.int32(np.uint32(2654435761).astype(np.int32))
_P2 = np.int32(805459861)
_RES = 128.0

_NC = 2
_NS = 16
_NW = _NC * _NS
_LANES = 16

_CHUNK = 256
_NBUF = 4
_RBLK = 8


def _reformat_body(*, n_blocks):
    blk_per_w = n_blocks // _NW

    n_steps = blk_per_w // _RBLK

    def body(tv_ref, t8_ref, *refs):
        bufs = tuple(
            (refs[i], refs[4 + i], refs[8 + i], refs[12 + i]) for i in range(4)
        )
        wid = lax.axis_index("s") * _NC + lax.axis_index("c")
        iota = lax.iota(jnp.int32, _LANES)

        def src_slice(kk):
            return tv_ref.at[pl.ds((wid * blk_per_w + kk * _RBLK) * 512, _RBLK * 512)]

        def dst_slice(kk):
            return t8_ref.at[pl.ds((wid * blk_per_w + kk * _RBLK) * 128, _RBLK * 128)]

        def fire_in(kk, bi):
            pltpu.async_copy(src_slice(kk), bufs[bi][0], bufs[bi][2])

        def compute(kk, bi):
            in_v, out_v, isem, osem = bufs[bi]
            pltpu.make_async_copy(src_slice(kk), in_v, isem).wait()

            @pl.when(kk >= 4)
            def _wait_prev_out():
                pltpu.make_async_copy(out_v, dst_slice(kk - 4), osem).wait()

            for b in range(_RBLK):
                for f in range(4):
                    col = jnp.full((_LANES,), f, jnp.int32)
                    for k in range(8):
                        v = in_v[pl.ds(b * 512 + f * 128 + k * 16, _LANES)]
                        row = b * 128 + k * 16 + iota
                        plsc.store_scatter(out_v, [row, col], v)
            pltpu.async_copy(out_v, dst_slice(kk), bufs[bi][3])

        for k in range(3):
            fire_in(k, k)

        def quad_body(p, _):
            for j in range(4):
                k = p * 4 + j
                ka = k + 3

                @pl.when(ka < n_steps)
                def _pf():
                    fire_in(ka, (j + 3) % 4)

                compute(k, j)
            return _

        lax.fori_loop(0, n_steps // 4, quad_body, None)
        for j in range(4):
            k = n_steps - 4 + j
            pltpu.make_async_copy(bufs[j][1], dst_slice(k), bufs[j][3]).wait()

    return body


def _main_body(*, n_points, mask):
    pts_per_w = n_points // _NW
    n_chunks = pts_per_w // _CHUNK

    def body(x_ref, table_ref, out_ref, *refs):
        nper = 5
        bufs = []
        for i in range(_NBUF):
            r = refs[i * nper:(i + 1) * nper]
            bufs.append((r[0], r[1], r[2], r[3], r[4]))
        s0 = _NBUF * nper
        gsems = refs[s0:s0 + _NBUF]
        xsems = refs[s0 + _NBUF:s0 + 2 * _NBUF]
        osems = refs[s0 + 2 * _NBUF:s0 + 3 * _NBUF]
        wid = lax.axis_index("s") * _NC + lax.axis_index("c")
        iota = lax.iota(jnp.int32, _LANES)

        def x_slice(kk):
            blk0 = wid * (pts_per_w // 128) + kk * (_CHUNK // 128)
            return x_ref.at[pl.ds(blk0 * 512, (_CHUNK // 128) * 512)]

        def out_slice(kk):
            blk0 = wid * (pts_per_w // 128) + kk * (_CHUNK // 128)
            return out_ref.at[pl.ds(blk0 * 512, (_CHUNK // 128) * 512)]

        def fire_x(kk, bi):
            pltpu.async_copy(x_slice(kk), bufs[bi][0], xsems[bi])

        def phase_a(kk, bi, guard_next):
            x_v, w_v, out_v, idx_all, rows_all = bufs[bi]
            pltpu.make_async_copy(x_slice(kk), x_v, xsems[bi]).wait()

            def hash_body(g, _):
                off = (lax.shift_right_logical(g, 3)) * 512 + jnp.bitwise_and(g, 7) * 16
                x0 = x_v[pl.ds(off, _LANES)]
                x1 = x_v[pl.ds(off + 128, _LANES)]
                x2 = x_v[pl.ds(off + 256, _LANES)]
                s0 = x0 * _RES
                s1 = x1 * _RES
                s2 = x2 * _RES
                f0 = s0.astype(jnp.int32)
                f1 = s1.astype(jnp.int32)
                f2 = s2.astype(jnp.int32)
                f0f = f0.astype(jnp.float32)
                f1f = f1.astype(jnp.float32)
                f2f = f2.astype(jnp.float32)
                d0 = s0 - f0f
                d1 = s1 - f1f
                d2 = s2 - f2f
                c0 = f0 + (s0 > f0f).astype(jnp.int32)
                c1 = f1 + (s1 > f1f).astype(jnp.int32)
                c2 = f2 + (s2 > f2f).astype(jnp.int32)
                bf = f1 * _P1
                bc = c1 * _P1
                gf = f2 * _P2
                gc = c2 * _P2
                t00 = f0 ^ bf
                t10 = c0 ^ bf
                t01 = f0 ^ bc
                t11 = c0 ^ bc
                base = g * _LANES
                hs = (
                    (t00 ^ gf) & mask,
                    (t10 ^ gf) & mask,
                    (t01 ^ gf) & mask,
                    (t00 ^ gc) & mask,
                    (t11 ^ gf) & mask,
                    (t10 ^ gc) & mask,
                    (t01 ^ gc) & mask,
                    (t11 ^ gc) & mask,
                )
                for c in range(8):
                    idx_all[pl.ds(base + c * _CHUNK, _LANES)] = hs[c]
                sl = pl.ds(base, _LANES)
                u0 = 1.0 - d0
                u1 = 1.0 - d1
                u2 = 1.0 - d2
                a00 = u0 * u1
                a10 = d0 * u1
                a01 = u0 * d1
                a11 = d0 * d1
                w_v[0, sl] = a00 * u2
                w_v[1, sl] = a10 * u2
                w_v[2, sl] = a01 * u2
                w_v[3, sl] = a00 * d2
                w_v[4, sl] = a11 * u2
                w_v[5, sl] = a10 * d2
                w_v[6, sl] = a01 * d2
                w_v[7, sl] = a11 * d2
                return _

            lax.fori_loop(0, _CHUNK // _LANES, hash_body, None)

            pltpu.async_copy(table_ref.at[idx_all], rows_all, gsems[bi])

            nbi = (bi + 1) % _NBUF
            if guard_next:
                @pl.when(kk + 1 < n_chunks)
                def _px():
                    fire_x(kk + 1, nbi)
            else:
                fire_x(kk + 1, nbi)

        def phase_b(kk, bi):
            x_v, w_v, out_v, idx_all, rows_all = bufs[bi]
            pltpu.make_async_copy(
                table_ref.at[idx_all], rows_all, gsems[bi]
            ).wait()

            @pl.when(kk >= _NBUF)
            def _wait_prev_out():
                pltpu.make_async_copy(out_v, out_slice(kk - _NBUF), osems[bi]).wait()

            def interp_body(g, _):
                e16 = g * _LANES + iota
                ws = [w_v[c, pl.ds(g * _LANES, _LANES)] for c in range(8)]
                ecs = [e16 + c * _CHUNK for c in range(8)]
                off = (lax.shift_right_logical(g, 3)) * 512 + jnp.bitwise_and(g, 7) * 16
                for f in range(4):
                    fsp = jnp.full((_LANES,), f, jnp.int32)
                    acc = jnp.zeros((_LANES,), jnp.float32)
                    for c in range(8):
                        r = plsc.load_gather(rows_all, [ecs[c], fsp])
                        acc = acc + ws[c] * r
                    out_v[pl.ds(off + f * 128, _LANES)] = acc
                return _

            lax.fori_loop(0, _CHUNK // _LANES, interp_body, None)

            pltpu.async_copy(out_v, out_slice(kk), osems[bi])

        fire_x(0, 0)
        for k in range(_NBUF - 1):
            phase_a(k, k, guard_next=False)

        def quad_body(p, _):
            for j in range(_NBUF):
                k = p * _NBUF + j
                ka = k + _NBUF - 1

                @pl.when(ka < n_chunks)
                def _pa():
                    phase_a(ka, (j + _NBUF - 1) % _NBUF, guard_next=True)

                phase_b(k, j)
            return _

        lax.fori_loop(0, n_chunks // _NBUF, quad_body, None)
        for j in range(_NBUF):
            k = n_chunks - _NBUF + j
            pltpu.make_async_copy(bufs[j][2], out_slice(k), osems[j]).wait()

    return body


@jax.jit
def _run(x_view, table_view):
    n_points = x_view.shape[0] // 4
    size = table_view.shape[0] // 4
    n_blocks = size // 128
    mask = np.int32(size - 1)
    mesh = plsc.VectorSubcoreMesh(
        core_axis_name="c", subcore_axis_name="s", num_cores=_NC, num_subcores=_NS
    )
    params = pltpu.CompilerParams(
        needs_layout_passes=False, use_tc_tiling_on_sc=False
    )
    reformat = pl.kernel(
        _reformat_body(n_blocks=n_blocks),
        out_type=jax.ShapeDtypeStruct((size, 8), jnp.float32),
        mesh=mesh,
        scratch_types=(
            [pltpu.VMEM((_RBLK * 512,), jnp.float32) for _ in range(4)]
            + [pltpu.VMEM((_RBLK * 128, 8), jnp.float32) for _ in range(4)]
            + [pltpu.SemaphoreType.DMA for _ in range(8)]
        ),
        compiler_params=params,
    )
    table8 = reformat(table_view)
    main = pl.kernel(
        _main_body(n_points=n_points, mask=mask),
        out_type=jax.ShapeDtypeStruct((n_points * 4,), jnp.float32),
        mesh=mesh,
        scratch_types=(
            [
                pltpu.VMEM(((_CHUNK // 128) * 512,), jnp.float32),
                pltpu.VMEM((8, _CHUNK), jnp.float32),
                pltpu.VMEM(((_CHUNK // 128) * 512,), jnp.float32),
            ]
            + [
                pltpu.VMEM((8 * _CHUNK,), jnp.int32),
                pltpu.VMEM((8 * _CHUNK, 8), jnp.float32),
            ]
        ) * _NBUF
        + [pltpu.SemaphoreType.DMA for _ in range(3 * _NBUF)],
        compiler_params=params,
    )
    return main(x_view, table8)


def kernel(x, hashtable):
    n = x.reshape(-1, 3).shape[0]
    size = hashtable.shape[0]
    x4 = jnp.concatenate([x.reshape(n, 3), jnp.zeros((n, 1), jnp.float32)], axis=1)
    x_view = x4.reshape(n // 128, 128, 4).transpose(0, 2, 1).reshape(-1)
    t_view = hashtable.reshape(size // 128, 128, 4).transpose(0, 2, 1).reshape(-1)
    out_flat = _run(x_view, t_view)
    out = out_flat.reshape(n // 128, 4, 128).transpose(0, 2, 1).reshape(n, 4)
    return out.reshape(x.shape[:-1] + (4,))


# speedup vs baseline: 1.0261x; 1.0261x over previous
import functools

import jax
import jax.numpy as jnp
import numpy as np
from jax import lax
from jax.experimental import pallas as pl
from jax.experimental.pallas import tpu as pltpu
from jax.experimental.pallas import tpu_sc as plsc

_P1 = np.int32(np.uint32(2654435761).astype(np.int32))
_P2 = np.int32(805459861)
_RES = 128.0

_NC = 2
_NS = 16
_NW = _NC * _NS
_LANES = 16

_CHUNK = 256
_NBUF = 4
_RBLK = 8


def _reformat_body(*, n_blocks):
    blk_per_w = n_blocks // _NW

    n_steps = blk_per_w // _RBLK

    def body(tv_ref, t8_ref, *refs):
        bufs = ((refs[0], refs[2], refs[4], refs[6]), (refs[1], refs[3], refs[5], refs[7]))
        wid = lax.axis_index("s") * _NC + lax.axis_index("c")
        iota = lax.iota(jnp.int32, _LANES)

        def src_slice(kk):
            return tv_ref.at[pl.ds((wid * blk_per_w + kk * _RBLK) * 512, _RBLK * 512)]

        def dst_slice(kk):
            return t8_ref.at[pl.ds((wid * blk_per_w + kk * _RBLK) * 128, _RBLK * 128)]

        def fire_in(kk, bi):
            pltpu.async_copy(src_slice(kk), bufs[bi][0], bufs[bi][2])

        def compute(kk, bi):
            in_v, out_v, isem, osem = bufs[bi]
            pltpu.make_async_copy(src_slice(kk), in_v, isem).wait()

            @pl.when(kk >= 2)
            def _wait_prev_out():
                pltpu.make_async_copy(out_v, dst_slice(kk - 2), osem).wait()

            for b in range(_RBLK):
                for f in range(4):
                    col = jnp.full((_LANES,), f, jnp.int32)
                    for k in range(8):
                        v = in_v[pl.ds(b * 512 + f * 128 + k * 16, _LANES)]
                        row = b * 128 + k * 16 + iota
                        plsc.store_scatter(out_v, [row, col], v)
            pltpu.async_copy(out_v, dst_slice(kk), bufs[bi][3])

        fire_in(0, 0)

        def pair_body(p, _):
            k0 = p * 2
            fire_in(k0 + 1, 1)
            compute(k0, 0)

            @pl.when(k0 + 2 < n_steps)
            def _prefetch():
                fire_in(k0 + 2, 0)

            compute(k0 + 1, 1)
            return _

        lax.fori_loop(0, n_steps // 2, pair_body, None)
        pltpu.make_async_copy(bufs[0][1], dst_slice(n_steps - 2), bufs[0][3]).wait()
        pltpu.make_async_copy(bufs[1][1], dst_slice(n_steps - 1), bufs[1][3]).wait()

    return body


def _main_body(*, n_points, mask):
    pts_per_w = n_points // _NW
    n_chunks = pts_per_w // _CHUNK

    def body(x_ref, table_ref, out_ref, *refs):
        nper = 5
        bufs = []
        for i in range(_NBUF):
            r = refs[i * nper:(i + 1) * nper]
            bufs.append((r[0], r[1], r[2], r[3], r[4]))
        s0 = _NBUF * nper
        gsems = refs[s0:s0 + _NBUF]
        xsems = refs[s0 + _NBUF:s0 + 2 * _NBUF]
        osems = refs[s0 + 2 * _NBUF:s0 + 3 * _NBUF]
        wid = lax.axis_index("s") * _NC + lax.axis_index("c")
        iota = lax.iota(jnp.int32, _LANES)

        def x_slice(kk):
            blk0 = wid * (pts_per_w // 128) + kk * (_CHUNK // 128)
            return x_ref.at[pl.ds(blk0 * 512, (_CHUNK // 128) * 512)]

        def out_slice(kk):
            blk0 = wid * (pts_per_w // 128) + kk * (_CHUNK // 128)
            return out_ref.at[pl.ds(blk0 * 512, (_CHUNK // 128) * 512)]

        def fire_x(kk, bi):
            pltpu.async_copy(x_slice(kk), bufs[bi][0], xsems[bi])

        def phase_a(kk, bi, guard_next):
            x_v, w_v, out_v, idx_all, rows_all = bufs[bi]
            pltpu.make_async_copy(x_slice(kk), x_v, xsems[bi]).wait()

            def hash_body(g, _):
                off = (lax.shift_right_logical(g, 3)) * 512 + jnp.bitwise_and(g, 7) * 16
                x0 = x_v[pl.ds(off, _LANES)]
                x1 = x_v[pl.ds(off + 128, _LANES)]
                x2 = x_v[pl.ds(off + 256, _LANES)]
                s0 = x0 * _RES
                s1 = x1 * _RES
                s2 = x2 * _RES
                f0 = s0.astype(jnp.int32)
                f1 = s1.astype(jnp.int32)
                f2 = s2.astype(jnp.int32)
                f0f = f0.astype(jnp.float32)
                f1f = f1.astype(jnp.float32)
                f2f = f2.astype(jnp.float32)
                d0 = s0 - f0f
                d1 = s1 - f1f
                d2 = s2 - f2f
                c0 = f0 + (s0 > f0f).astype(jnp.int32)
                c1 = f1 + (s1 > f1f).astype(jnp.int32)
                c2 = f2 + (s2 > f2f).astype(jnp.int32)
                bf = f1 * _P1
                bc = c1 * _P1
                gf = f2 * _P2
                gc = c2 * _P2
                t00 = f0 ^ bf
                t10 = c0 ^ bf
                t01 = f0 ^ bc
                t11 = c0 ^ bc
                base = g * _LANES
                hs = (
                    (t00 ^ gf) & mask,
                    (t10 ^ gf) & mask,
                    (t01 ^ gf) & mask,
                    (t00 ^ gc) & mask,
                    (t11 ^ gf) & mask,
                    (t10 ^ gc) & mask,
                    (t01 ^ gc) & mask,
                    (t11 ^ gc) & mask,
                )
                for c in range(8):
                    idx_all[pl.ds(base + c * _CHUNK, _LANES)] = hs[c]
                sl = pl.ds(base, _LANES)
                u0 = 1.0 - d0
                u1 = 1.0 - d1
                u2 = 1.0 - d2
                a00 = u0 * u1
                a10 = d0 * u1
                a01 = u0 * d1
                a11 = d0 * d1
                w_v[0, sl] = a00 * u2
                w_v[1, sl] = a10 * u2
                w_v[2, sl] = a01 * u2
                w_v[3, sl] = a00 * d2
                w_v[4, sl] = a11 * u2
                w_v[5, sl] = a10 * d2
                w_v[6, sl] = a01 * d2
                w_v[7, sl] = a11 * d2
                return _

            lax.fori_loop(0, _CHUNK // _LANES, hash_body, None)

            pltpu.async_copy(table_ref.at[idx_all], rows_all, gsems[bi])

            nbi = (bi + 1) % _NBUF
            if guard_next:
                @pl.when(kk + 1 < n_chunks)
                def _px():
                    fire_x(kk + 1, nbi)
            else:
                fire_x(kk + 1, nbi)

        def phase_b(kk, bi):
            x_v, w_v, out_v, idx_all, rows_all = bufs[bi]
            pltpu.make_async_copy(
                table_ref.at[idx_all], rows_all, gsems[bi]
            ).wait()

            @pl.when(kk >= _NBUF)
            def _wait_prev_out():
                pltpu.make_async_copy(out_v, out_slice(kk - _NBUF), osems[bi]).wait()

            def interp_body(g, _):
                e16 = g * _LANES + iota
                ws = [w_v[c, pl.ds(g * _LANES, _LANES)] for c in range(8)]
                ecs = [e16 + c * _CHUNK for c in range(8)]
                off = (lax.shift_right_logical(g, 3)) * 512 + jnp.bitwise_and(g, 7) * 16
                for f in range(4):
                    fsp = jnp.full((_LANES,), f, jnp.int32)
                    acc = jnp.zeros((_LANES,), jnp.float32)
                    for c in range(8):
                        r = plsc.load_gather(rows_all, [ecs[c], fsp])
                        acc = acc + ws[c] * r
                    out_v[pl.ds(off + f * 128, _LANES)] = acc
                return _

            lax.fori_loop(0, _CHUNK // _LANES, interp_body, None)

            pltpu.async_copy(out_v, out_slice(kk), osems[bi])

        fire_x(0, 0)
        for k in range(_NBUF - 1):
            phase_a(k, k, guard_next=False)

        def quad_body(p, _):
            for j in range(_NBUF):
                k = p * _NBUF + j
                ka = k + _NBUF - 1

                @pl.when(ka < n_chunks)
                def _pa():
                    phase_a(ka, (j + _NBUF - 1) % _NBUF, guard_next=True)

                phase_b(k, j)
            return _

        lax.fori_loop(0, n_chunks // _NBUF, quad_body, None)
        for j in range(_NBUF):
            k = n_chunks - _NBUF + j
            pltpu.make_async_copy(bufs[j][2], out_slice(k), osems[j]).wait()

    return body


@jax.jit
def _run(x_view, table_view):
    n_points = x_view.shape[0] // 4
    size = table_view.shape[0] // 4
    n_blocks = size // 128
    mask = np.int32(size - 1)
    mesh = plsc.VectorSubcoreMesh(
        core_axis_name="c", subcore_axis_name="s", num_cores=_NC, num_subcores=_NS
    )
    params = pltpu.CompilerParams(
        needs_layout_passes=False, use_tc_tiling_on_sc=False
    )
    reformat = pl.kernel(
        _reformat_body(n_blocks=n_blocks),
        out_type=jax.ShapeDtypeStruct((size, 8), jnp.float32),
        mesh=mesh,
        scratch_types=[
            pltpu.VMEM((_RBLK * 512,), jnp.float32),
            pltpu.VMEM((_RBLK * 512,), jnp.float32),
            pltpu.VMEM((_RBLK * 128, 8), jnp.float32),
            pltpu.VMEM((_RBLK * 128, 8), jnp.float32),
            pltpu.SemaphoreType.DMA,
            pltpu.SemaphoreType.DMA,
            pltpu.SemaphoreType.DMA,
            pltpu.SemaphoreType.DMA,
        ],
        compiler_params=params,
    )
    table8 = reformat(table_view)
    main = pl.kernel(
        _main_body(n_points=n_points, mask=mask),
        out_type=jax.ShapeDtypeStruct((n_points * 4,), jnp.float32),
        mesh=mesh,
        scratch_types=(
            [
                pltpu.VMEM(((_CHUNK // 128) * 512,), jnp.float32),
                pltpu.VMEM((8, _CHUNK), jnp.float32),
                pltpu.VMEM(((_CHUNK // 128) * 512,), jnp.float32),
            ]
            + [
                pltpu.VMEM((8 * _CHUNK,), jnp.int32),
                pltpu.VMEM((8 * _CHUNK, 8), jnp.float32),
            ]
        ) * _NBUF
        + [pltpu.SemaphoreType.DMA for _ in range(3 * _NBUF)],
        compiler_params=params,
    )
    return main(x_view, table8)


def kernel(x, hashtable):
    n = x.reshape(-1, 3).shape[0]
    size = hashtable.shape[0]
    x4 = jnp.concatenate([x.reshape(n, 3), jnp.zeros((n, 1), jnp.float32)], axis=1)
    x_view = x4.reshape(n // 128, 128, 4).transpose(0, 2, 1).reshape(-1)
    t_view = hashtable.reshape(size // 128, 128, 4).transpose(0, 2, 1).reshape(-1)
    out_flat = _run(x_view, t_view)
    out = out_flat.reshape(n // 128, 4, 128).transpose(0, 2, 1).reshape(n, 4)
    return out.reshape(x.shape[:-1] + (4,))
